# baseline (speedup 1.0000x reference)
import jax
import jax.numpy as jnp
from jax import lax
from jax.experimental import pallas as pl
from jax.experimental.pallas import tpu as pltpu
from jax.experimental.pallas import tpu_sc as plsc

N_NODES = 100000
E_EDGES = 1600000
IN_DIM = 6
HID = 32
HALF = 16

LANES = 128
E_PAD = 1638400
E_ROWS = E_PAD // LANES
N_TILES = 16
GROUP_ROWS = 8

NB = 25
BSHIFT = 12
BSIZE = 4096
BMASK = BSIZE - 1
ACC_L = BSIZE + LANES
OUT_ROWS = NB * BSIZE

N_WORKERS = 32
EDGES_PER_WORKER = E_PAD // N_WORKERS
PART_GROUPS = EDGES_PER_WORKER // (GROUP_ROWS * LANES)
SLOT_ROWS = EDGES_PER_WORKER // LANES + 8
STAGE_CAP = 1280

DEG_ROWS = 100352
DEG_STRIPE = DEG_ROWS // N_TILES
DEG_STRIPE_CHUNKS = DEG_STRIPE // LANES
DEG_ROWS_PER_CORE = E_ROWS // 2
DEG_ROWS_PER_TILE = DEG_ROWS_PER_CORE // N_TILES
DEG_GROUPS = DEG_ROWS_PER_TILE // GROUP_ROWS

_MESH = plsc.VectorSubcoreMesh(core_axis_name="c", subcore_axis_name="s")


def _part_body(src_hbm, dst_hbm, osrc_hbm, odst_hbm, cnt_hbm, *scratch):
    sbuf, dbuf = scratch[0], scratch[1]
    st_s = scratch[2:2 + NB]
    st_d = scratch[2 + NB:2 + 2 * NB]
    cbuf = scratch[2 + 2 * NB]

    c = lax.axis_index("c")
    t = lax.axis_index("s")
    w = c * 16 + t
    e0w = w * EDGES_PER_WORKER
    iota16 = lax.iota(jnp.int32, 16)

    def flush(b, cnt, rows):
        nf = cnt >> 7

        def fl(k, _):
            pltpu.sync_copy(st_s[b].at[pl.ds(k * LANES, LANES)],
                            osrc_hbm.at[b, w, rows + k])
            pltpu.sync_copy(st_d[b].at[pl.ds(k * LANES, LANES)],
                            odst_hbm.at[b, w, rows + k])
            return 0

        lax.fori_loop(0, nf, fl, 0)
        off = nf * LANES
        for i in range(8):
            sv = st_s[b][pl.ds(off + i * 16, 16)]
            dv = st_d[b][pl.ds(off + i * 16, 16)]
            st_s[b][pl.ds(i * 16, 16)] = sv
            st_d[b][pl.ds(i * 16, 16)] = dv
        return cnt - (nf << 7), rows + nf

    def group(g, carry):
        cnts = list(carry[:NB])
        rows = list(carry[NB:])
        pltpu.sync_copy(src_hbm.at[pl.ds(e0w + g * 1024, 1024)], sbuf)
        pltpu.sync_copy(dst_hbm.at[pl.ds(e0w + g * 1024, 1024)], dbuf)

        def lanes16(k, cc):
            sl = pl.ds(k * 16, 16)
            s16 = sbuf[sl]
            d16 = dbuf[sl]
            b16 = lax.shift_right_logical(d16, BSHIFT)
            dl16 = jnp.bitwise_and(d16, BMASK)
            out = []
            for b in range(NB):
                m = b16 == b
                plsc.store_compressed(st_s[b].at[pl.ds(cc[b], 16)], s16, mask=m)
                plsc.store_compressed(st_d[b].at[pl.ds(cc[b], 16)], dl16, mask=m)
                n = plsc.all_reduce_population_count(m)[0]
                out.append(cc[b] + n)
            return tuple(out)

        cnts = list(lax.fori_loop(0, 64, lanes16, tuple(cnts)))
        for b in range(NB):
            cnts[b], rows[b] = flush(b, cnts[b], rows[b])
        return (*cnts, *rows)

    zero = jnp.int32(0)
    carry = lax.fori_loop(0, PART_GROUPS, group, (zero,) * (2 * NB))
    cnts = list(carry[:NB])
    rows = list(carry[NB:])

    zero16 = jnp.zeros((16,), jnp.int32)
    garb16 = jnp.full((16,), BSIZE, jnp.int32)
    finals = []
    for b in range(NB):
        cnt, rws = cnts[b], rows[b]
        for i in range(8):
            st_s[b][pl.ds(cnt + i * 16, 16)] = zero16
            st_d[b][pl.ds(cnt + i * 16, 16)] = garb16
        pltpu.sync_copy(st_s[b].at[pl.ds(0, LANES)], osrc_hbm.at[b, w, rws])
        pltpu.sync_copy(st_d[b].at[pl.ds(0, LANES)], odst_hbm.at[b, w, rws])
        rws = rws + 1
        for i in range(8):
            st_s[b][pl.ds(i * 16, 16)] = zero16
            st_d[b][pl.ds(i * 16, 16)] = garb16
        nextra = (-rws) & 7

        def pad(k, _):
            pltpu.sync_copy(st_s[b].at[pl.ds(0, LANES)], osrc_hbm.at[b, w, rws + k])
            pltpu.sync_copy(st_d[b].at[pl.ds(0, LANES)], odst_hbm.at[b, w, rws + k])
            return 0

        lax.fori_loop(0, nextra, pad, 0)
        finals.append((rws + nextra) >> 3)

    lo = jnp.zeros((16,), jnp.int32)
    hi = jnp.zeros((16,), jnp.int32)
    for b in range(16):
        lo = jnp.where(iota16 == b, finals[b], lo)
    for b in range(16, NB):
        hi = jnp.where(iota16 == (b - 16), finals[b], hi)
    cbuf[pl.ds(0, 16)] = lo
    cbuf[pl.ds(16, 16)] = hi
    pltpu.sync_copy(cbuf, cnt_hbm.at[w])


_part = pl.kernel(
    _part_body,
    out_type=(
        jax.ShapeDtypeStruct((NB, N_WORKERS, SLOT_ROWS, LANES), jnp.int32),
        jax.ShapeDtypeStruct((NB, N_WORKERS, SLOT_ROWS, LANES), jnp.int32),
        jax.ShapeDtypeStruct((N_WORKERS, 32), jnp.int32),
    ),
    mesh=_MESH,
    scratch_types=(
        [pltpu.VMEM((GROUP_ROWS * LANES,), jnp.int32)] * 2
        + [pltpu.VMEM((STAGE_CAP,), jnp.int32)] * (2 * NB)
        + [pltpu.VMEM((32,), jnp.int32)]
    ),
    compiler_params=pltpu.CompilerParams(use_tc_tiling_on_sc=False,
                                         needs_layout_passes=False),
)


def _agg_body(osrc_hbm, odst_hbm, cnt_hbm, gt_hbm, out_hbm,
              sbuf, dbuf, rows, cnt_spm, cnt_smem, acc, gsem):
    c = lax.axis_index("c")
    t = lax.axis_index("s")

    pltpu.sync_copy(cnt_hbm, cnt_spm)
    pltpu.sync_copy(cnt_spm, cnt_smem)

    z16 = jnp.zeros((16,), jnp.float32)
    iota16 = lax.iota(jnp.int32, 16)
    gt_c = gt_hbm.at[c]

    for rep in range(2):
        b = t + 16 * rep

        @pl.when(b < NB)
        def _():
            def zr(r, _):
                acc[r, :] = z16
                return 0

            lax.fori_loop(0, ACC_L, zr, 0)

            def per_w(w, _):
                n8 = cnt_smem[w, b]

                def grp(r8, _):
                    pltpu.sync_copy(osrc_hbm.at[b, w, pl.ds(r8 * 8, 8)], sbuf)
                    pltpu.sync_copy(odst_hbm.at[b, w, pl.ds(r8 * 8, 8)], dbuf)
                    gathers = [
                        pltpu.async_copy(gt_c.at[sbuf.at[j]],
                                         rows.at[pl.ds(j * LANES, LANES)], gsem)
                        for j in range(GROUP_ROWS)
                    ]
                    for h in gathers:
                        h.wait()
                    for j in range(GROUP_ROWS):
                        def lg(k, _):
                            e16 = iota16 + (j * LANES + k * 16)
                            dl16 = dbuf[j, pl.ds(k * 16, 16)]
                            for f in range(HALF):
                                fs = jnp.full((16,), f, jnp.int32)
                                vals = plsc.load_gather(rows, [e16, fs])
                                plsc.addupdate_scatter(acc, [dl16, fs], vals)
                            return 0

                        lax.fori_loop(0, LANES // 16, lg, 0)
                    return 0

                lax.fori_loop(0, n8, grp, 0)
                return 0

            lax.fori_loop(0, N_WORKERS, per_w, 0)

            def wb(kk, _):
                pltpu.sync_copy(acc.at[pl.ds(kk * LANES, LANES)],
                                out_hbm.at[c].at[pl.ds(b * BSIZE + kk * LANES, LANES)])
                return 0

            lax.fori_loop(0, BSIZE // LANES, wb, 0)


_agg = pl.kernel(
    _agg_body,
    out_type=jax.ShapeDtypeStruct((2, OUT_ROWS, HALF), jnp.float32),
    mesh=_MESH,
    scratch_types=[
        pltpu.VMEM((GROUP_ROWS, LANES), jnp.int32),
        pltpu.VMEM((GROUP_ROWS, LANES), jnp.int32),
        pltpu.VMEM((GROUP_ROWS * LANES, HALF), jnp.float32),
        pltpu.VMEM_SHARED((N_WORKERS, 32), jnp.int32),
        pltpu.SMEM((N_WORKERS, 32), jnp.int32),
        pltpu.VMEM((ACC_L, HALF), jnp.float32),
        pltpu.SemaphoreType.DMA,
    ],
    compiler_params=pltpu.CompilerParams(use_tc_tiling_on_sc=False,
                                         needs_layout_passes=False),
)


def _deg_body(dst_hbm, out_hbm, dbuf, obuf, acc, ssem):
    c = lax.axis_index("c")
    t = lax.axis_index("s")

    one = jnp.ones((16,), jnp.float32)
    z = jnp.zeros((16,), jnp.float32)
    stripe0 = t * DEG_STRIPE

    def zb(i, _):
        obuf[pl.ds(i * 16, 16)] = z
        return 0

    lax.fori_loop(0, LANES // 16, zb, 0)

    def zacc(k, _):
        pltpu.sync_copy(obuf, acc.at[pl.ds(stripe0 + k * LANES, LANES)])
        return 0

    lax.fori_loop(0, DEG_STRIPE_CHUNKS, zacc, 0)

    def ofill(i, _):
        obuf[pl.ds(i * 16, 16)] = one
        return 0

    lax.fori_loop(0, LANES // 16, ofill, 0)
    plsc.subcore_barrier()

    row0 = c * DEG_ROWS_PER_CORE + t * DEG_ROWS_PER_TILE

    def group(g, _):
        r = row0 + g * GROUP_ROWS
        pltpu.sync_copy(dst_hbm.at[pl.ds(r, GROUP_ROWS)], dbuf)
        scatters = [
            pltpu.async_copy(obuf, acc.at[dbuf.at[j]], ssem, add=True)
            for j in range(GROUP_ROWS)
        ]
        for h in scatters:
            h.wait()
        return 0

    lax.fori_loop(0, DEG_GROUPS, group, 0)

    plsc.subcore_barrier()

    def wb(k, _):
        off = stripe0 + k * LANES
        pltpu.sync_copy(acc.at[pl.ds(off, LANES)], out_hbm.at[c].at[pl.ds(off, LANES)])
        return 0

    lax.fori_loop(0, DEG_STRIPE_CHUNKS, wb, 0)


_deg = pl.kernel(
    _deg_body,
    out_type=jax.ShapeDtypeStruct((2, DEG_ROWS), jnp.float32),
    mesh=_MESH,
    scratch_types=[
        pltpu.VMEM((GROUP_ROWS, LANES), jnp.int32),
        pltpu.VMEM((LANES,), jnp.float32),
        pltpu.VMEM_SHARED((DEG_ROWS,), jnp.float32),
        pltpu.SemaphoreType.DMA,
    ],
    compiler_params=pltpu.CompilerParams(use_tc_tiling_on_sc=False),
)


_BLK = 4096
_GRID = (N_NODES + _BLK - 1) // _BLK


def _pre_body(deg0_ref, deg1_ref, x_ref, w1_ref, dis_ref, h1_ref, gt_ref):
    deg = deg0_ref[...] + deg1_ref[...] + 1.0
    dis = lax.rsqrt(deg)
    h = jnp.dot(x_ref[...], w1_ref[...], preferred_element_type=jnp.float32)
    g = h * dis
    dis_ref[...] = dis
    h1_ref[...] = h
    gt_ref[0] = g[:, :HALF]
    gt_ref[1] = g[:, HALF:]


_pre = pl.pallas_call(
    _pre_body,
    grid=(_GRID,),
    in_specs=[
        pl.BlockSpec((_BLK, 1), lambda i: (i, 0)),
        pl.BlockSpec((_BLK, 1), lambda i: (i, 0)),
        pl.BlockSpec((_BLK, IN_DIM), lambda i: (i, 0)),
        pl.BlockSpec((IN_DIM, HID), lambda i: (0, 0)),
    ],
    out_specs=[
        pl.BlockSpec((_BLK, 1), lambda i: (i, 0)),
        pl.BlockSpec((_BLK, HID), lambda i: (i, 0)),
        pl.BlockSpec((2, _BLK, HALF), lambda i: (0, i, 0)),
    ],
    out_shape=[
        jax.ShapeDtypeStruct((N_NODES, 1), jnp.float32),
        jax.ShapeDtypeStruct((N_NODES, HID), jnp.float32),
        jax.ShapeDtypeStruct((2, N_NODES, HALF), jnp.float32),
    ],
)


def _mid_body(acc_ref, h1_ref, dis_ref, w2_ref, b1_ref, h2_ref, gt2_ref):
    dis = dis_ref[...]
    agg = jnp.concatenate([acc_ref[0], acc_ref[1]], axis=1)
    out1 = jnp.maximum(agg * dis + h1_ref[...] * (dis * dis) + b1_ref[...], 0.0)
    h2 = jnp.dot(out1, w2_ref[...], preferred_element_type=jnp.float32)
    g2 = h2 * dis
    h2_ref[...] = h2
    gt2_ref[0] = g2[:, :HALF]
    gt2_ref[1] = g2[:, HALF:]


_mid = pl.pallas_call(
    _mid_body,
    grid=(_GRID,),
    in_specs=[
        pl.BlockSpec((2, _BLK, HALF), lambda i: (0, i, 0)),
        pl.BlockSpec((_BLK, HID), lambda i: (i, 0)),
        pl.BlockSpec((_BLK, 1), lambda i: (i, 0)),
        pl.BlockSpec((HID, HID), lambda i: (0, 0)),
        pl.BlockSpec((1, HID), lambda i: (0, 0)),
    ],
    out_specs=[
        pl.BlockSpec((_BLK, HID), lambda i: (i, 0)),
        pl.BlockSpec((2, _BLK, HALF), lambda i: (0, i, 0)),
    ],
    out_shape=[
        jax.ShapeDtypeStruct((N_NODES, HID), jnp.float32),
        jax.ShapeDtypeStruct((2, N_NODES, HALF), jnp.float32),
    ],
)


def _post_body(acc_ref, h2_ref, dis_ref, b2_ref, wp_ref, bp_ref, mask_ref,
               logit_ref):
    dis = dis_ref[...]
    agg = jnp.concatenate([acc_ref[0], acc_ref[1]], axis=1)
    out2 = jnp.maximum(agg * dis + h2_ref[...] * (dis * dis) + b2_ref[...], 0.0)
    z = jnp.sum(out2 * wp_ref[...], axis=1, keepdims=True) + bp_ref[0, 0]
    logit_ref[...] = jnp.where(mask_ref[...] > 0, z, jnp.float32(-1e9))


_post = pl.pallas_call(
    _post_body,
    grid=(_GRID,),
    in_specs=[
        pl.BlockSpec((2, _BLK, HALF), lambda i: (0, i, 0)),
        pl.BlockSpec((_BLK, HID), lambda i: (i, 0)),
        pl.BlockSpec((_BLK, 1), lambda i: (i, 0)),
        pl.BlockSpec((1, HID), lambda i: (0, 0)),
        pl.BlockSpec((1, HID), lambda i: (0, 0)),
        pl.BlockSpec((1, 1), lambda i: (0, 0)),
        pl.BlockSpec((_BLK, 1), lambda i: (i, 0)),
    ],
    out_specs=[pl.BlockSpec((_BLK, 1), lambda i: (i, 0))],
    out_shape=[jax.ShapeDtypeStruct((N_NODES, 1), jnp.float32)],
)


def _softmax_body(x_ref, o_ref):
    x = x_ref[...]
    m = jnp.max(x)
    e = jnp.exp(x - m)
    o_ref[...] = e / jnp.sum(e)


_softmax = pl.pallas_call(
    _softmax_body,
    out_shape=jax.ShapeDtypeStruct((1, N_NODES), jnp.float32),
)


def kernel(x, edge_index, ready_mask, W1, b1, W2, b2, Wp, bp):
    src = edge_index[0]
    dst = edge_index[1]
    pad = E_PAD - E_EDGES
    src1d = jnp.concatenate([src, jnp.zeros((pad,), src.dtype)])
    dst1d = jnp.concatenate([dst, jnp.full((pad,), N_NODES, dst.dtype)])
    dstp = dst1d.reshape(E_ROWS, LANES)

    degout = _deg(dstp)
    deg0 = degout[0, :N_NODES].reshape(N_NODES, 1)
    deg1 = degout[1, :N_NODES].reshape(N_NODES, 1)

    osrc, odst, cnts = _part(src1d, dst1d)

    dis, h1, gt1 = _pre(deg0, deg1, x, W1)
    acc1 = _agg(osrc, odst, cnts, gt1)[:, :N_NODES]
    h2, gt2 = _mid(acc1, h1, dis, W2, b1.reshape(1, HID))
    acc2 = _agg(osrc, odst, cnts, gt2)[:, :N_NODES]

    mask = ready_mask.astype(jnp.int32).reshape(N_NODES, 1)
    (logits,) = _post(acc2, h2, dis, b2.reshape(1, HID),
                      Wp.reshape(1, HID), bp.reshape(1, 1), mask)
    probs = _softmax(logits.reshape(1, N_NODES))
    return probs.reshape(N_NODES)


# speedup vs baseline: 1.1181x; 1.1181x over previous
import jax
import jax.numpy as jnp
from jax import lax
from jax.experimental import pallas as pl
from jax.experimental.pallas import tpu as pltpu
from jax.experimental.pallas import tpu_sc as plsc

N_NODES = 100000
E_EDGES = 1600000
IN_DIM = 6
HID = 32
HALF = 16

LANES = 128
E_PAD = 1638400
E_ROWS = E_PAD // LANES
N_TILES = 16
GROUP_ROWS = 8

NB = 25
BSHIFT = 12
BSIZE = 4096
BMASK = BSIZE - 1
ACC_L = BSIZE + LANES
OUT_ROWS = NB * BSIZE

N_WORKERS = 32
EDGES_PER_WORKER = E_PAD // N_WORKERS
PART_GROUPS = EDGES_PER_WORKER // (GROUP_ROWS * LANES)
SLOT_ROWS = EDGES_PER_WORKER // LANES + 8
STAGE_CAP = 1280

DEG_ROWS = 100352
DEG_STRIPE = DEG_ROWS // N_TILES
DEG_STRIPE_CHUNKS = DEG_STRIPE // LANES
DEG_ROWS_PER_CORE = E_ROWS // 2
DEG_ROWS_PER_TILE = DEG_ROWS_PER_CORE // N_TILES
DEG_GROUPS = DEG_ROWS_PER_TILE // GROUP_ROWS

_MESH = plsc.VectorSubcoreMesh(core_axis_name="c", subcore_axis_name="s")


def _part_body(src_hbm, dst_hbm, osrc_hbm, odst_hbm, cnt_hbm, *scratch):
    sbuf, dbuf = scratch[0], scratch[1]
    st_s = scratch[2:2 + NB]
    st_d = scratch[2 + NB:2 + 2 * NB]
    cbuf = scratch[2 + 2 * NB]

    c = lax.axis_index("c")
    t = lax.axis_index("s")
    w = c * 16 + t
    e0w = w * EDGES_PER_WORKER
    iota16 = lax.iota(jnp.int32, 16)

    def flush(b, cnt, rows):
        nf = cnt >> 7

        def fl(k, _):
            pltpu.sync_copy(st_s[b].at[pl.ds(k * LANES, LANES)],
                            osrc_hbm.at[b, w, rows + k])
            pltpu.sync_copy(st_d[b].at[pl.ds(k * LANES, LANES)],
                            odst_hbm.at[b, w, rows + k])
            return 0

        lax.fori_loop(0, nf, fl, 0)
        off = nf * LANES
        for i in range(8):
            sv = st_s[b][pl.ds(off + i * 16, 16)]
            dv = st_d[b][pl.ds(off + i * 16, 16)]
            st_s[b][pl.ds(i * 16, 16)] = sv
            st_d[b][pl.ds(i * 16, 16)] = dv
        return cnt - (nf << 7), rows + nf

    def group(g, carry):
        cnts = list(carry[:NB])
        rows = list(carry[NB:])
        pltpu.sync_copy(src_hbm.at[pl.ds(e0w + g * 1024, 1024)], sbuf)
        pltpu.sync_copy(dst_hbm.at[pl.ds(e0w + g * 1024, 1024)], dbuf)

        def lanes16(k, cc):
            sl = pl.ds(k * 16, 16)
            s16 = sbuf[sl]
            d16 = dbuf[sl]
            b16 = lax.shift_right_logical(d16, BSHIFT)
            dl16 = jnp.bitwise_and(d16, BMASK)
            out = []
            for b in range(NB):
                m = b16 == b
                plsc.store_compressed(st_s[b].at[pl.ds(cc[b], 16)], s16, mask=m)
                plsc.store_compressed(st_d[b].at[pl.ds(cc[b], 16)], dl16, mask=m)
                n = plsc.all_reduce_population_count(m)[0]
                out.append(cc[b] + n)
            return tuple(out)

        cnts = list(lax.fori_loop(0, 64, lanes16, tuple(cnts)))
        for b in range(NB):
            cnts[b], rows[b] = flush(b, cnts[b], rows[b])
        return (*cnts, *rows)

    zero = jnp.int32(0)
    carry = lax.fori_loop(0, PART_GROUPS, group, (zero,) * (2 * NB))
    cnts = list(carry[:NB])
    rows = list(carry[NB:])

    zero16 = jnp.zeros((16,), jnp.int32)
    garb16 = jnp.full((16,), BSIZE, jnp.int32)
    finals = []
    for b in range(NB):
        cnt, rws = cnts[b], rows[b]
        for i in range(8):
            st_s[b][pl.ds(cnt + i * 16, 16)] = zero16
            st_d[b][pl.ds(cnt + i * 16, 16)] = garb16
        pltpu.sync_copy(st_s[b].at[pl.ds(0, LANES)], osrc_hbm.at[b, w, rws])
        pltpu.sync_copy(st_d[b].at[pl.ds(0, LANES)], odst_hbm.at[b, w, rws])
        rws = rws + 1
        for i in range(8):
            st_s[b][pl.ds(i * 16, 16)] = zero16
            st_d[b][pl.ds(i * 16, 16)] = garb16
        nextra = (-rws) & 7

        def pad(k, _):
            pltpu.sync_copy(st_s[b].at[pl.ds(0, LANES)], osrc_hbm.at[b, w, rws + k])
            pltpu.sync_copy(st_d[b].at[pl.ds(0, LANES)], odst_hbm.at[b, w, rws + k])
            return 0

        lax.fori_loop(0, nextra, pad, 0)
        finals.append((rws + nextra) >> 3)

    lo = jnp.zeros((16,), jnp.int32)
    hi = jnp.zeros((16,), jnp.int32)
    for b in range(16):
        lo = jnp.where(iota16 == b, finals[b], lo)
    for b in range(16, NB):
        hi = jnp.where(iota16 == (b - 16), finals[b], hi)
    cbuf[pl.ds(0, 16)] = lo
    cbuf[pl.ds(16, 16)] = hi
    pltpu.sync_copy(cbuf, cnt_hbm.at[w])


_part = pl.kernel(
    _part_body,
    out_type=(
        jax.ShapeDtypeStruct((NB, N_WORKERS, SLOT_ROWS, LANES), jnp.int32),
        jax.ShapeDtypeStruct((NB, N_WORKERS, SLOT_ROWS, LANES), jnp.int32),
        jax.ShapeDtypeStruct((N_WORKERS, 32), jnp.int32),
    ),
    mesh=_MESH,
    scratch_types=(
        [pltpu.VMEM((GROUP_ROWS * LANES,), jnp.int32)] * 2
        + [pltpu.VMEM((STAGE_CAP,), jnp.int32)] * (2 * NB)
        + [pltpu.VMEM((32,), jnp.int32)]
    ),
    compiler_params=pltpu.CompilerParams(use_tc_tiling_on_sc=False,
                                         needs_layout_passes=False),
)


_FS = None


def _agg_body(osrc_hbm, odst_hbm, cnt_hbm, gt_hbm, out_hbm,
              sbuf, dbuf, rows, cnt_spm, cnt_smem, acc, gsem):
    global _FS
    _FS = [jnp.full((16,), f, jnp.int32) for f in range(HALF)]
    c = lax.axis_index("c")
    t = lax.axis_index("s")

    pltpu.sync_copy(cnt_hbm, cnt_spm)
    pltpu.sync_copy(cnt_spm, cnt_smem)

    z16 = jnp.zeros((16,), jnp.float32)
    iota16 = lax.iota(jnp.int32, 16)
    gt_c = gt_hbm.at[c]

    for rep in range(2):
        b = t + 16 * rep

        @pl.when(b < NB)
        def _():
            def zr(r, _):
                acc[r, :] = z16
                return 0

            lax.fori_loop(0, ACC_L, zr, 0)

            def per_w(w, _):
                n8 = cnt_smem[w, b]

                def grp(r8, _):
                    pltpu.sync_copy(osrc_hbm.at[b, w, pl.ds(r8 * 8, 8)], sbuf)
                    pltpu.sync_copy(odst_hbm.at[b, w, pl.ds(r8 * 8, 8)], dbuf)
                    gathers = [
                        pltpu.async_copy(gt_c.at[sbuf.at[j]],
                                         rows.at[pl.ds(j * LANES, LANES)], gsem)
                        for j in range(GROUP_ROWS)
                    ]
                    for h in gathers:
                        h.wait()
                    for j in range(GROUP_ROWS):
                        for k in range(LANES // 16):
                            e16 = iota16 + (j * LANES + k * 16)
                            dl16 = dbuf[j, pl.ds(k * 16, 16)]
                            vals = [plsc.load_gather(rows, [e16, _FS[f]])
                                    for f in range(HALF)]
                            for f in range(HALF):
                                plsc.addupdate_scatter(acc, [dl16, _FS[f]], vals[f])
                    return 0

                lax.fori_loop(0, n8, grp, 0)
                return 0

            lax.fori_loop(0, N_WORKERS, per_w, 0)

            def wb(kk, _):
                pltpu.sync_copy(acc.at[pl.ds(kk * LANES, LANES)],
                                out_hbm.at[c].at[pl.ds(b * BSIZE + kk * LANES, LANES)])
                return 0

            lax.fori_loop(0, BSIZE // LANES, wb, 0)


_agg = pl.kernel(
    _agg_body,
    out_type=jax.ShapeDtypeStruct((2, OUT_ROWS, HALF), jnp.float32),
    mesh=_MESH,
    scratch_types=[
        pltpu.VMEM((GROUP_ROWS, LANES), jnp.int32),
        pltpu.VMEM((GROUP_ROWS, LANES), jnp.int32),
        pltpu.VMEM((GROUP_ROWS * LANES, HALF), jnp.float32),
        pltpu.VMEM_SHARED((N_WORKERS, 32), jnp.int32),
        pltpu.SMEM((N_WORKERS, 32), jnp.int32),
        pltpu.VMEM((ACC_L, HALF), jnp.float32),
        pltpu.SemaphoreType.DMA,
    ],
    compiler_params=pltpu.CompilerParams(use_tc_tiling_on_sc=False,
                                         needs_layout_passes=False,
                                         disable_bounds_checks=True),
)


def _deg_body(dst_hbm, out_hbm, dbuf, obuf, acc, ssem):
    c = lax.axis_index("c")
    t = lax.axis_index("s")

    one = jnp.ones((16,), jnp.float32)
    z = jnp.zeros((16,), jnp.float32)
    stripe0 = t * DEG_STRIPE

    def zb(i, _):
        obuf[pl.ds(i * 16, 16)] = z
        return 0

    lax.fori_loop(0, LANES // 16, zb, 0)

    def zacc(k, _):
        pltpu.sync_copy(obuf, acc.at[pl.ds(stripe0 + k * LANES, LANES)])
        return 0

    lax.fori_loop(0, DEG_STRIPE_CHUNKS, zacc, 0)

    def ofill(i, _):
        obuf[pl.ds(i * 16, 16)] = one
        return 0

    lax.fori_loop(0, LANES // 16, ofill, 0)
    plsc.subcore_barrier()

    row0 = c * DEG_ROWS_PER_CORE + t * DEG_ROWS_PER_TILE

    def group(g, _):
        r = row0 + g * GROUP_ROWS
        pltpu.sync_copy(dst_hbm.at[pl.ds(r, GROUP_ROWS)], dbuf)
        scatters = [
            pltpu.async_copy(obuf, acc.at[dbuf.at[j]], ssem, add=True)
            for j in range(GROUP_ROWS)
        ]
        for h in scatters:
            h.wait()
        return 0

    lax.fori_loop(0, DEG_GROUPS, group, 0)

    plsc.subcore_barrier()

    def wb(k, _):
        off = stripe0 + k * LANES
        pltpu.sync_copy(acc.at[pl.ds(off, LANES)], out_hbm.at[c].at[pl.ds(off, LANES)])
        return 0

    lax.fori_loop(0, DEG_STRIPE_CHUNKS, wb, 0)


_deg = pl.kernel(
    _deg_body,
    out_type=jax.ShapeDtypeStruct((2, DEG_ROWS), jnp.float32),
    mesh=_MESH,
    scratch_types=[
        pltpu.VMEM((GROUP_ROWS, LANES), jnp.int32),
        pltpu.VMEM((LANES,), jnp.float32),
        pltpu.VMEM_SHARED((DEG_ROWS,), jnp.float32),
        pltpu.SemaphoreType.DMA,
    ],
    compiler_params=pltpu.CompilerParams(use_tc_tiling_on_sc=False),
)


_BLK = 4096
_GRID = (N_NODES + _BLK - 1) // _BLK


def _pre_body(deg0_ref, deg1_ref, x_ref, w1_ref, dis_ref, h1_ref, gt_ref):
    deg = deg0_ref[...] + deg1_ref[...] + 1.0
    dis = lax.rsqrt(deg)
    h = jnp.dot(x_ref[...], w1_ref[...], preferred_element_type=jnp.float32)
    g = h * dis
    dis_ref[...] = dis
    h1_ref[...] = h
    gt_ref[0] = g[:, :HALF]
    gt_ref[1] = g[:, HALF:]


_pre = pl.pallas_call(
    _pre_body,
    grid=(_GRID,),
    in_specs=[
        pl.BlockSpec((_BLK, 1), lambda i: (i, 0)),
        pl.BlockSpec((_BLK, 1), lambda i: (i, 0)),
        pl.BlockSpec((_BLK, IN_DIM), lambda i: (i, 0)),
        pl.BlockSpec((IN_DIM, HID), lambda i: (0, 0)),
    ],
    out_specs=[
        pl.BlockSpec((_BLK, 1), lambda i: (i, 0)),
        pl.BlockSpec((_BLK, HID), lambda i: (i, 0)),
        pl.BlockSpec((2, _BLK, HALF), lambda i: (0, i, 0)),
    ],
    out_shape=[
        jax.ShapeDtypeStruct((N_NODES, 1), jnp.float32),
        jax.ShapeDtypeStruct((N_NODES, HID), jnp.float32),
        jax.ShapeDtypeStruct((2, N_NODES, HALF), jnp.float32),
    ],
)


def _mid_body(acc_ref, h1_ref, dis_ref, w2_ref, b1_ref, h2_ref, gt2_ref):
    dis = dis_ref[...]
    agg = jnp.concatenate([acc_ref[0], acc_ref[1]], axis=1)
    out1 = jnp.maximum(agg * dis + h1_ref[...] * (dis * dis) + b1_ref[...], 0.0)
    h2 = jnp.dot(out1, w2_ref[...], preferred_element_type=jnp.float32)
    g2 = h2 * dis
    h2_ref[...] = h2
    gt2_ref[0] = g2[:, :HALF]
    gt2_ref[1] = g2[:, HALF:]


_mid = pl.pallas_call(
    _mid_body,
    grid=(_GRID,),
    in_specs=[
        pl.BlockSpec((2, _BLK, HALF), lambda i: (0, i, 0)),
        pl.BlockSpec((_BLK, HID), lambda i: (i, 0)),
        pl.BlockSpec((_BLK, 1), lambda i: (i, 0)),
        pl.BlockSpec((HID, HID), lambda i: (0, 0)),
        pl.BlockSpec((1, HID), lambda i: (0, 0)),
    ],
    out_specs=[
        pl.BlockSpec((_BLK, HID), lambda i: (i, 0)),
        pl.BlockSpec((2, _BLK, HALF), lambda i: (0, i, 0)),
    ],
    out_shape=[
        jax.ShapeDtypeStruct((N_NODES, HID), jnp.float32),
        jax.ShapeDtypeStruct((2, N_NODES, HALF), jnp.float32),
    ],
)


def _post_body(acc_ref, h2_ref, dis_ref, b2_ref, wp_ref, bp_ref, mask_ref,
               logit_ref):
    dis = dis_ref[...]
    agg = jnp.concatenate([acc_ref[0], acc_ref[1]], axis=1)
    out2 = jnp.maximum(agg * dis + h2_ref[...] * (dis * dis) + b2_ref[...], 0.0)
    z = jnp.sum(out2 * wp_ref[...], axis=1, keepdims=True) + bp_ref[0, 0]
    logit_ref[...] = jnp.where(mask_ref[...] > 0, z, jnp.float32(-1e9))


_post = pl.pallas_call(
    _post_body,
    grid=(_GRID,),
    in_specs=[
        pl.BlockSpec((2, _BLK, HALF), lambda i: (0, i, 0)),
        pl.BlockSpec((_BLK, HID), lambda i: (i, 0)),
        pl.BlockSpec((_BLK, 1), lambda i: (i, 0)),
        pl.BlockSpec((1, HID), lambda i: (0, 0)),
        pl.BlockSpec((1, HID), lambda i: (0, 0)),
        pl.BlockSpec((1, 1), lambda i: (0, 0)),
        pl.BlockSpec((_BLK, 1), lambda i: (i, 0)),
    ],
    out_specs=[pl.BlockSpec((_BLK, 1), lambda i: (i, 0))],
    out_shape=[jax.ShapeDtypeStruct((N_NODES, 1), jnp.float32)],
)


def _softmax_body(x_ref, o_ref):
    x = x_ref[...]
    m = jnp.max(x)
    e = jnp.exp(x - m)
    o_ref[...] = e / jnp.sum(e)


_softmax = pl.pallas_call(
    _softmax_body,
    out_shape=jax.ShapeDtypeStruct((1, N_NODES), jnp.float32),
)


def kernel(x, edge_index, ready_mask, W1, b1, W2, b2, Wp, bp):
    src = edge_index[0]
    dst = edge_index[1]
    pad = E_PAD - E_EDGES
    src1d = jnp.concatenate([src, jnp.zeros((pad,), src.dtype)])
    dst1d = jnp.concatenate([dst, jnp.full((pad,), N_NODES, dst.dtype)])
    dstp = dst1d.reshape(E_ROWS, LANES)

    degout = _deg(dstp)
    deg0 = degout[0, :N_NODES].reshape(N_NODES, 1)
    deg1 = degout[1, :N_NODES].reshape(N_NODES, 1)

    osrc, odst, cnts = _part(src1d, dst1d)

    dis, h1, gt1 = _pre(deg0, deg1, x, W1)
    acc1 = _agg(osrc, odst, cnts, gt1)[:, :N_NODES]
    h2, gt2 = _mid(acc1, h1, dis, W2, b1.reshape(1, HID))
    acc2 = _agg(osrc, odst, cnts, gt2)[:, :N_NODES]

    mask = ready_mask.astype(jnp.int32).reshape(N_NODES, 1)
    (logits,) = _post(acc2, h2, dis, b2.reshape(1, HID),
                      Wp.reshape(1, HID), bp.reshape(1, 1), mask)
    probs = _softmax(logits.reshape(1, N_NODES))
    return probs.reshape(N_NODES)


# speedup vs baseline: 4.0183x; 3.5940x over previous
import jax
import jax.numpy as jnp
from jax import lax
from jax.experimental import pallas as pl
from jax.experimental.pallas import tpu as pltpu
from jax.experimental.pallas import tpu_sc as plsc

N_NODES = 100000
E_EDGES = 1600000
IN_DIM = 6
HID = 32
HALF = 16

LANES = 128
E_PAD = 1638400
E_ROWS = E_PAD // LANES
N_TILES = 16

M_SPLIT = 50176
ACC_ROWS = 51200
STRIPE = ACC_ROWS // N_TILES
STRIPE_CHUNKS = STRIPE // LANES

GROUP_ROWS = 8
ROWS_PER_TILE = E_ROWS // N_TILES
N_GROUPS = ROWS_PER_TILE // GROUP_ROWS

DEG_ROWS = 100352
DEG_STRIPE = DEG_ROWS // N_TILES
DEG_STRIPE_CHUNKS = DEG_STRIPE // LANES
DEG_ROWS_PER_CORE = E_ROWS // 2
DEG_ROWS_PER_TILE = DEG_ROWS_PER_CORE // N_TILES
DEG_GROUPS = DEG_ROWS_PER_TILE // GROUP_ROWS

_MESH = plsc.VectorSubcoreMesh(core_axis_name="c", subcore_axis_name="s")


N_WORKERS = 32
PART_ROWS_PER_WORKER = E_ROWS // N_WORKERS
PART_GROUPS = PART_ROWS_PER_WORKER // GROUP_ROWS
SLOT_ROWS = PART_ROWS_PER_WORKER + 8
STAGE_CAP = 1280
GARBAGE = M_SPLIT


def _part_body(src_hbm, dst_hbm, osrc_hbm, odst_hbm, cnt_hbm,
               sbuf, dbuf, st_s0, st_d0, st_s1, st_d1, cbuf):
    c = lax.axis_index("c")
    t = lax.axis_index("s")
    w = c * 16 + t
    row0 = w * PART_ROWS_PER_WORKER
    stages = ((st_s0, st_d0), (st_s1, st_d1))

    def flush(p, st_s, st_d, cnt, rows):
        nf = cnt >> 7

        def fl(k, _):
            pltpu.sync_copy(st_s.at[pl.ds(k * LANES, LANES)],
                            osrc_hbm.at[p, w, rows + k])
            pltpu.sync_copy(st_d.at[pl.ds(k * LANES, LANES)],
                            odst_hbm.at[p, w, rows + k])
            return 0

        lax.fori_loop(0, nf, fl, 0)
        off = nf * LANES
        for i in range(8):
            sv = st_s[pl.ds(off + i * 16, 16)]
            dv = st_d[pl.ds(off + i * 16, 16)]
            st_s[pl.ds(i * 16, 16)] = sv
            st_d[pl.ds(i * 16, 16)] = dv
        return cnt - (nf << 7), rows + nf

    def group(g, carry):
        cnt0, rows0, cnt1, rows1 = carry
        r = row0 + g * GROUP_ROWS
        pltpu.sync_copy(src_hbm.at[pl.ds(r, GROUP_ROWS)], sbuf)
        pltpu.sync_copy(dst_hbm.at[pl.ds(r, GROUP_ROWS)], dbuf)
        for j in range(GROUP_ROWS):
            def lanes16(k, cc):
                c0, c1 = cc
                sl = pl.ds(k * 16, 16)
                s16 = sbuf[j, sl]
                d16 = dbuf[j, sl]
                m0 = d16 < M_SPLIT
                m1 = jnp.logical_not(m0)
                d16b = d16 - M_SPLIT
                plsc.store_compressed(st_s0.at[pl.ds(c0, 16)], s16, mask=m0)
                plsc.store_compressed(st_d0.at[pl.ds(c0, 16)], d16, mask=m0)
                plsc.store_compressed(st_s1.at[pl.ds(c1, 16)], s16, mask=m1)
                plsc.store_compressed(st_d1.at[pl.ds(c1, 16)], d16b, mask=m1)
                n0 = plsc.all_reduce_population_count(m0)[0]
                return c0 + n0, c1 + (16 - n0)

            cnt0, cnt1 = lax.fori_loop(0, LANES // 16, lanes16, (cnt0, cnt1))
        cnt0, rows0 = flush(0, st_s0, st_d0, cnt0, rows0)
        cnt1, rows1 = flush(1, st_s1, st_d1, cnt1, rows1)
        return cnt0, rows0, cnt1, rows1

    cnt0, rows0, cnt1, rows1 = lax.fori_loop(
        0, PART_GROUPS, group, (jnp.int32(0), jnp.int32(0), jnp.int32(0), jnp.int32(0)))

    zero16 = jnp.zeros((16,), jnp.int32)
    garb16 = jnp.full((16,), GARBAGE, jnp.int32)
    final = []
    for p, (st_s, st_d) in enumerate(stages):
        cnt, rows = (cnt0, rows0) if p == 0 else (cnt1, rows1)
        for i in range(8):
            st_s[pl.ds(cnt + i * 16, 16)] = zero16
            st_d[pl.ds(cnt + i * 16, 16)] = garb16
        pltpu.sync_copy(st_s.at[pl.ds(0, LANES)], osrc_hbm.at[p, w, rows])
        pltpu.sync_copy(st_d.at[pl.ds(0, LANES)], odst_hbm.at[p, w, rows])
        rows = rows + 1
        for i in range(8):
            st_s[pl.ds(i * 16, 16)] = zero16
            st_d[pl.ds(i * 16, 16)] = garb16
        nextra = (-rows) & 7

        def pad(k, _):
            pltpu.sync_copy(st_s.at[pl.ds(0, LANES)], osrc_hbm.at[p, w, rows + k])
            pltpu.sync_copy(st_d.at[pl.ds(0, LANES)], odst_hbm.at[p, w, rows + k])
            return 0

        lax.fori_loop(0, nextra, pad, 0)
        final.append((rows + nextra) >> 3)

    idx16 = lax.iota(jnp.int32, 16)
    cvec = jnp.where(idx16 == 0, final[0], jnp.where(idx16 == 1, final[1], 0))
    cbuf[...] = cvec
    pltpu.sync_copy(cbuf, cnt_hbm.at[w])


_part = pl.kernel(
    _part_body,
    out_type=(
        jax.ShapeDtypeStruct((2, N_WORKERS, SLOT_ROWS, LANES), jnp.int32),
        jax.ShapeDtypeStruct((2, N_WORKERS, SLOT_ROWS, LANES), jnp.int32),
        jax.ShapeDtypeStruct((N_WORKERS, 16), jnp.int32),
    ),
    mesh=_MESH,
    scratch_types=[
        pltpu.VMEM((GROUP_ROWS, LANES), jnp.int32),
        pltpu.VMEM((GROUP_ROWS, LANES), jnp.int32),
        pltpu.VMEM((STAGE_CAP,), jnp.int32),
        pltpu.VMEM((STAGE_CAP,), jnp.int32),
        pltpu.VMEM((STAGE_CAP,), jnp.int32),
        pltpu.VMEM((STAGE_CAP,), jnp.int32),
        pltpu.VMEM((16,), jnp.int32),
    ],
    compiler_params=pltpu.CompilerParams(use_tc_tiling_on_sc=False,
                                         needs_layout_passes=False),
)


def _agg_body(osrc_hbm, odst_hbm, cnt_hbm, gt_hbm, out_hbm,
              sbuf, dbuf, rows, zbuf, cnt_spm, cnt_smem, acc, gsem, ssem):
    c = lax.axis_index("c")
    t = lax.axis_index("s")

    pltpu.sync_copy(cnt_hbm, cnt_spm)
    pltpu.sync_copy(cnt_spm, cnt_smem)

    z = jnp.zeros((16,), jnp.float32)

    def zfill(i, _):
        zbuf[i, :] = z
        return 0

    lax.fori_loop(0, LANES, zfill, 0)

    gt_c = gt_hbm.at[c]
    stripe0 = t * STRIPE

    for p in (0, 1):
        def zacc(k, _):
            pltpu.sync_copy(zbuf, acc.at[pl.ds(stripe0 + k * LANES, LANES)])
            return 0

        lax.fori_loop(0, STRIPE_CHUNKS, zacc, 0)
        plsc.subcore_barrier()

        for si in range(2):
            w = t * 2 + si
            n8 = cnt_smem[w, p]

            def group(g, _):
                r = g * GROUP_ROWS
                pltpu.sync_copy(osrc_hbm.at[p, w, pl.ds(r, GROUP_ROWS)], sbuf)
                pltpu.sync_copy(odst_hbm.at[p, w, pl.ds(r, GROUP_ROWS)], dbuf)
                gathers = [
                    pltpu.async_copy(gt_c.at[sbuf.at[j]], rows.at[j], gsem)
                    for j in range(GROUP_ROWS)
                ]
                for h in gathers:
                    h.wait()
                scatters = [
                    pltpu.async_copy(rows.at[j], acc.at[dbuf.at[j]], ssem, add=True)
                    for j in range(GROUP_ROWS)
                ]
                for h in scatters:
                    h.wait()
                return 0

            lax.fori_loop(0, n8, group, 0)

        plsc.subcore_barrier()

        def wb(k, _):
            off = stripe0 + k * LANES
            pltpu.sync_copy(acc.at[pl.ds(off, LANES)],
                            out_hbm.at[c].at[pl.ds(p * M_SPLIT + off, LANES)])
            return 0

        lax.fori_loop(0, STRIPE_CHUNKS, wb, 0)
        plsc.subcore_barrier()


_agg = pl.kernel(
    _agg_body,
    out_type=jax.ShapeDtypeStruct((2, M_SPLIT + ACC_ROWS, HALF), jnp.float32),
    mesh=_MESH,
    scratch_types=[
        pltpu.VMEM((GROUP_ROWS, LANES), jnp.int32),
        pltpu.VMEM((GROUP_ROWS, LANES), jnp.int32),
        pltpu.VMEM((GROUP_ROWS, LANES, HALF), jnp.float32),
        pltpu.VMEM((LANES, HALF), jnp.float32),
        pltpu.VMEM_SHARED((N_WORKERS, 16), jnp.int32),
        pltpu.SMEM((N_WORKERS, 16), jnp.int32),
        pltpu.VMEM_SHARED((ACC_ROWS, HALF), jnp.float32),
        pltpu.SemaphoreType.DMA,
        pltpu.SemaphoreType.DMA,
    ],
    compiler_params=pltpu.CompilerParams(use_tc_tiling_on_sc=False),
)


def _deg_body(dst_hbm, out_hbm, dbuf, obuf, acc, ssem):
    c = lax.axis_index("c")
    t = lax.axis_index("s")

    one = jnp.ones((16,), jnp.float32)
    z = jnp.zeros((16,), jnp.float32)
    stripe0 = t * DEG_STRIPE

    def zb(i, _):
        obuf[pl.ds(i * 16, 16)] = z
        return 0

    lax.fori_loop(0, LANES // 16, zb, 0)

    def zacc(k, _):
        pltpu.sync_copy(obuf, acc.at[pl.ds(stripe0 + k * LANES, LANES)])
        return 0

    lax.fori_loop(0, DEG_STRIPE_CHUNKS, zacc, 0)

    def ofill(i, _):
        obuf[pl.ds(i * 16, 16)] = one
        return 0

    lax.fori_loop(0, LANES // 16, ofill, 0)
    plsc.subcore_barrier()

    row0 = c * DEG_ROWS_PER_CORE + t * DEG_ROWS_PER_TILE

    def group(g, _):
        r = row0 + g * GROUP_ROWS
        pltpu.sync_copy(dst_hbm.at[pl.ds(r, GROUP_ROWS)], dbuf)
        scatters = [
            pltpu.async_copy(obuf, acc.at[dbuf.at[j]], ssem, add=True)
            for j in range(GROUP_ROWS)
        ]
        for h in scatters:
            h.wait()
        return 0

    lax.fori_loop(0, DEG_GROUPS, group, 0)

    plsc.subcore_barrier()

    def wb(k, _):
        off = stripe0 + k * LANES
        pltpu.sync_copy(acc.at[pl.ds(off, LANES)], out_hbm.at[c].at[pl.ds(off, LANES)])
        return 0

    lax.fori_loop(0, DEG_STRIPE_CHUNKS, wb, 0)


_deg = pl.kernel(
    _deg_body,
    out_type=jax.ShapeDtypeStruct((2, DEG_ROWS), jnp.float32),
    mesh=_MESH,
    scratch_types=[
        pltpu.VMEM((GROUP_ROWS, LANES), jnp.int32),
        pltpu.VMEM((LANES,), jnp.float32),
        pltpu.VMEM_SHARED((DEG_ROWS,), jnp.float32),
        pltpu.SemaphoreType.DMA,
    ],
)


_BLK = 4096
_GRID = (N_NODES + _BLK - 1) // _BLK


def _pre_body(deg0_ref, deg1_ref, x_ref, w1_ref, dis_ref, h1_ref, gt_ref):
    deg = deg0_ref[...] + deg1_ref[...] + 1.0
    dis = lax.rsqrt(deg)
    h = jnp.dot(x_ref[...], w1_ref[...], preferred_element_type=jnp.float32)
    g = h * dis
    dis_ref[...] = dis
    h1_ref[...] = h
    gt_ref[0] = g[:, :HALF]
    gt_ref[1] = g[:, HALF:]


_pre = pl.pallas_call(
    _pre_body,
    grid=(_GRID,),
    in_specs=[
        pl.BlockSpec((_BLK, 1), lambda i: (i, 0)),
        pl.BlockSpec((_BLK, 1), lambda i: (i, 0)),
        pl.BlockSpec((_BLK, IN_DIM), lambda i: (i, 0)),
        pl.BlockSpec((IN_DIM, HID), lambda i: (0, 0)),
    ],
    out_specs=[
        pl.BlockSpec((_BLK, 1), lambda i: (i, 0)),
        pl.BlockSpec((_BLK, HID), lambda i: (i, 0)),
        pl.BlockSpec((2, _BLK, HALF), lambda i: (0, i, 0)),
    ],
    out_shape=[
        jax.ShapeDtypeStruct((N_NODES, 1), jnp.float32),
        jax.ShapeDtypeStruct((N_NODES, HID), jnp.float32),
        jax.ShapeDtypeStruct((2, N_NODES, HALF), jnp.float32),
    ],
)


def _mid_body(acc_ref, h1_ref, dis_ref, w2_ref, b1_ref, h2_ref, gt2_ref):
    dis = dis_ref[...]
    agg = jnp.concatenate([acc_ref[0], acc_ref[1]], axis=1)
    out1 = jnp.maximum(agg * dis + h1_ref[...] * (dis * dis) + b1_ref[...], 0.0)
    h2 = jnp.dot(out1, w2_ref[...], preferred_element_type=jnp.float32)
    g2 = h2 * dis
    h2_ref[...] = h2
    gt2_ref[0] = g2[:, :HALF]
    gt2_ref[1] = g2[:, HALF:]


_mid = pl.pallas_call(
    _mid_body,
    grid=(_GRID,),
    in_specs=[
        pl.BlockSpec((2, _BLK, HALF), lambda i: (0, i, 0)),
        pl.BlockSpec((_BLK, HID), lambda i: (i, 0)),
        pl.BlockSpec((_BLK, 1), lambda i: (i, 0)),
        pl.BlockSpec((HID, HID), lambda i: (0, 0)),
        pl.BlockSpec((1, HID), lambda i: (0, 0)),
    ],
    out_specs=[
        pl.BlockSpec((_BLK, HID), lambda i: (i, 0)),
        pl.BlockSpec((2, _BLK, HALF), lambda i: (0, i, 0)),
    ],
    out_shape=[
        jax.ShapeDtypeStruct((N_NODES, HID), jnp.float32),
        jax.ShapeDtypeStruct((2, N_NODES, HALF), jnp.float32),
    ],
)


def _post_body(acc_ref, h2_ref, dis_ref, b2_ref, wp_ref, bp_ref, mask_ref,
               logit_ref):
    dis = dis_ref[...]
    agg = jnp.concatenate([acc_ref[0], acc_ref[1]], axis=1)
    out2 = jnp.maximum(agg * dis + h2_ref[...] * (dis * dis) + b2_ref[...], 0.0)
    z = jnp.sum(out2 * wp_ref[...], axis=1, keepdims=True) + bp_ref[0, 0]
    logit_ref[...] = jnp.where(mask_ref[...] > 0, z, jnp.float32(-1e9))


_post = pl.pallas_call(
    _post_body,
    grid=(_GRID,),
    in_specs=[
        pl.BlockSpec((2, _BLK, HALF), lambda i: (0, i, 0)),
        pl.BlockSpec((_BLK, HID), lambda i: (i, 0)),
        pl.BlockSpec((_BLK, 1), lambda i: (i, 0)),
        pl.BlockSpec((1, HID), lambda i: (0, 0)),
        pl.BlockSpec((1, HID), lambda i: (0, 0)),
        pl.BlockSpec((1, 1), lambda i: (0, 0)),
        pl.BlockSpec((_BLK, 1), lambda i: (i, 0)),
    ],
    out_specs=[pl.BlockSpec((_BLK, 1), lambda i: (i, 0))],
    out_shape=[jax.ShapeDtypeStruct((N_NODES, 1), jnp.float32)],
)


def _softmax_body(x_ref, o_ref):
    x = x_ref[...]
    m = jnp.max(x)
    e = jnp.exp(x - m)
    o_ref[...] = e / jnp.sum(e)


_softmax = pl.pallas_call(
    _softmax_body,
    out_shape=jax.ShapeDtypeStruct((1, N_NODES), jnp.float32),
)


def kernel(x, edge_index, ready_mask, W1, b1, W2, b2, Wp, bp):
    src = edge_index[0]
    dst = edge_index[1]
    pad = E_PAD - E_EDGES
    srcp = jnp.concatenate([src, jnp.zeros((pad,), src.dtype)]).reshape(E_ROWS, LANES)
    dstp = jnp.concatenate([dst, jnp.full((pad,), N_NODES, dst.dtype)]).reshape(E_ROWS, LANES)

    degout = _deg(dstp)
    deg0 = degout[0, :N_NODES].reshape(N_NODES, 1)
    deg1 = degout[1, :N_NODES].reshape(N_NODES, 1)

    osrc, odst, cnts = _part(srcp, dstp)

    dis, h1, gt1 = _pre(deg0, deg1, x, W1)
    acc1 = _agg(osrc, odst, cnts, gt1)
    h2, gt2 = _mid(acc1, h1, dis, W2, b1.reshape(1, HID))
    acc2 = _agg(osrc, odst, cnts, gt2)

    mask = ready_mask.astype(jnp.int32).reshape(N_NODES, 1)
    (logits,) = _post(acc2, h2, dis, b2.reshape(1, HID),
                      Wp.reshape(1, HID), bp.reshape(1, 1), mask)
    probs = _softmax(logits.reshape(1, N_NODES))
    return probs.reshape(N_NODES)


# speedup vs baseline: 4.2630x; 1.0609x over previous
import jax
import jax.numpy as jnp
from jax import lax
from jax.experimental import pallas as pl
from jax.experimental.pallas import tpu as pltpu
from jax.experimental.pallas import tpu_sc as plsc

N_NODES = 100000
E_EDGES = 1600000
IN_DIM = 6
HID = 32
HALF = 16

LANES = 128
E_PAD = 1638400
E_ROWS = E_PAD // LANES
N_TILES = 16

M_SPLIT = 50176
ACC_ROWS = 51200
STRIPE = ACC_ROWS // N_TILES
STRIPE_CHUNKS = STRIPE // LANES

GROUP_ROWS = 8
ROWS_PER_TILE = E_ROWS // N_TILES
N_GROUPS = ROWS_PER_TILE // GROUP_ROWS

DEG_ROWS = 100352
DEG_STRIPE = DEG_ROWS // N_TILES
DEG_STRIPE_CHUNKS = DEG_STRIPE // LANES
DEG_ROWS_PER_CORE = E_ROWS // 2
DEG_ROWS_PER_TILE = DEG_ROWS_PER_CORE // N_TILES
DEG_GROUPS = DEG_ROWS_PER_TILE // GROUP_ROWS

_MESH = plsc.VectorSubcoreMesh(core_axis_name="c", subcore_axis_name="s")


N_WORKERS = 32
PART_ROWS_PER_WORKER = E_ROWS // N_WORKERS
PART_GROUPS = PART_ROWS_PER_WORKER // GROUP_ROWS
SLOT_ROWS = PART_ROWS_PER_WORKER + 8
STAGE_CAP = 1280
GARBAGE = M_SPLIT


def _part_body(src_hbm, dst_hbm, osrc_hbm, odst_hbm, cnt_hbm,
               sbuf, dbuf, st_s0, st_d0, st_s1, st_d1, cbuf):
    c = lax.axis_index("c")
    t = lax.axis_index("s")
    w = c * 16 + t
    row0 = w * PART_ROWS_PER_WORKER
    stages = ((st_s0, st_d0), (st_s1, st_d1))

    def flush(p, st_s, st_d, cnt, rows):
        nf = cnt >> 7

        def fl(k, _):
            pltpu.sync_copy(st_s.at[pl.ds(k * LANES, LANES)],
                            osrc_hbm.at[p, w, rows + k])
            pltpu.sync_copy(st_d.at[pl.ds(k * LANES, LANES)],
                            odst_hbm.at[p, w, rows + k])
            return 0

        lax.fori_loop(0, nf, fl, 0)
        off = nf * LANES
        for i in range(8):
            sv = st_s[pl.ds(off + i * 16, 16)]
            dv = st_d[pl.ds(off + i * 16, 16)]
            st_s[pl.ds(i * 16, 16)] = sv
            st_d[pl.ds(i * 16, 16)] = dv
        return cnt - (nf << 7), rows + nf

    def group(g, carry):
        cnt0, rows0, cnt1, rows1 = carry
        r = row0 + g * GROUP_ROWS
        pltpu.sync_copy(src_hbm.at[pl.ds(r, GROUP_ROWS)], sbuf)
        pltpu.sync_copy(dst_hbm.at[pl.ds(r, GROUP_ROWS)], dbuf)
        for j in range(GROUP_ROWS):
            def lanes16(k, cc):
                c0, c1 = cc
                sl = pl.ds(k * 16, 16)
                s16 = sbuf[j, sl]
                d16 = dbuf[j, sl]
                m0 = d16 < M_SPLIT
                m1 = jnp.logical_not(m0)
                d16b = d16 - M_SPLIT
                plsc.store_compressed(st_s0.at[pl.ds(c0, 16)], s16, mask=m0)
                plsc.store_compressed(st_d0.at[pl.ds(c0, 16)], d16, mask=m0)
                plsc.store_compressed(st_s1.at[pl.ds(c1, 16)], s16, mask=m1)
                plsc.store_compressed(st_d1.at[pl.ds(c1, 16)], d16b, mask=m1)
                n0 = plsc.all_reduce_population_count(m0)[0]
                return c0 + n0, c1 + (16 - n0)

            cnt0, cnt1 = lax.fori_loop(0, LANES // 16, lanes16, (cnt0, cnt1))
        cnt0, rows0 = flush(0, st_s0, st_d0, cnt0, rows0)
        cnt1, rows1 = flush(1, st_s1, st_d1, cnt1, rows1)
        return cnt0, rows0, cnt1, rows1

    cnt0, rows0, cnt1, rows1 = lax.fori_loop(
        0, PART_GROUPS, group, (jnp.int32(0), jnp.int32(0), jnp.int32(0), jnp.int32(0)))

    zero16 = jnp.zeros((16,), jnp.int32)
    garb16 = jnp.full((16,), GARBAGE, jnp.int32)
    final = []
    for p, (st_s, st_d) in enumerate(stages):
        cnt, rows = (cnt0, rows0) if p == 0 else (cnt1, rows1)
        for i in range(8):
            st_s[pl.ds(cnt + i * 16, 16)] = zero16
            st_d[pl.ds(cnt + i * 16, 16)] = garb16
        pltpu.sync_copy(st_s.at[pl.ds(0, LANES)], osrc_hbm.at[p, w, rows])
        pltpu.sync_copy(st_d.at[pl.ds(0, LANES)], odst_hbm.at[p, w, rows])
        rows = rows + 1
        for i in range(8):
            st_s[pl.ds(i * 16, 16)] = zero16
            st_d[pl.ds(i * 16, 16)] = garb16
        nextra = (-rows) & 7

        def pad(k, _):
            pltpu.sync_copy(st_s.at[pl.ds(0, LANES)], osrc_hbm.at[p, w, rows + k])
            pltpu.sync_copy(st_d.at[pl.ds(0, LANES)], odst_hbm.at[p, w, rows + k])
            return 0

        lax.fori_loop(0, nextra, pad, 0)
        final.append((rows + nextra) >> 3)

    idx16 = lax.iota(jnp.int32, 16)
    cvec = jnp.where(idx16 == 0, final[0], jnp.where(idx16 == 1, final[1], 0))
    cbuf[...] = cvec
    pltpu.sync_copy(cbuf, cnt_hbm.at[w])


_part = pl.kernel(
    _part_body,
    out_type=(
        jax.ShapeDtypeStruct((2, N_WORKERS, SLOT_ROWS, LANES), jnp.int32),
        jax.ShapeDtypeStruct((2, N_WORKERS, SLOT_ROWS, LANES), jnp.int32),
        jax.ShapeDtypeStruct((N_WORKERS, 16), jnp.int32),
    ),
    mesh=_MESH,
    scratch_types=[
        pltpu.VMEM((GROUP_ROWS, LANES), jnp.int32),
        pltpu.VMEM((GROUP_ROWS, LANES), jnp.int32),
        pltpu.VMEM((STAGE_CAP,), jnp.int32),
        pltpu.VMEM((STAGE_CAP,), jnp.int32),
        pltpu.VMEM((STAGE_CAP,), jnp.int32),
        pltpu.VMEM((STAGE_CAP,), jnp.int32),
        pltpu.VMEM((16,), jnp.int32),
    ],
    compiler_params=pltpu.CompilerParams(use_tc_tiling_on_sc=False,
                                         needs_layout_passes=False),
)


def _agg_body(osrc_hbm, odst_hbm, cnt_hbm, gt_hbm, out_hbm,
              sbuf, dbuf, rows, zbuf, cnt_spm, cnt_smem, acc, gsem, ssem):
    c = lax.axis_index("c")
    t = lax.axis_index("s")

    pltpu.sync_copy(cnt_hbm, cnt_spm)
    pltpu.sync_copy(cnt_spm, cnt_smem)

    z = jnp.zeros((16,), jnp.float32)

    def zfill(i, _):
        zbuf[i, :] = z
        return 0

    lax.fori_loop(0, LANES, zfill, 0)

    gt_c = gt_hbm.at[c]
    stripe0 = t * STRIPE

    for p in (0, 1):
        def zacc(k, _):
            pltpu.sync_copy(zbuf, acc.at[pl.ds(stripe0 + k * LANES, LANES)])
            return 0

        lax.fori_loop(0, STRIPE_CHUNKS, zacc, 0)
        plsc.subcore_barrier()

        for si in range(2):
            w = t * 2 + si
            n8 = cnt_smem[w, p]

            def group(g, _):
                r = g * GROUP_ROWS
                pltpu.sync_copy(osrc_hbm.at[p, w, pl.ds(r, GROUP_ROWS)], sbuf)
                pltpu.sync_copy(odst_hbm.at[p, w, pl.ds(r, GROUP_ROWS)], dbuf)
                gathers = [
                    pltpu.async_copy(gt_c.at[sbuf.at[j]], rows.at[j], gsem)
                    for j in range(GROUP_ROWS)
                ]
                scatters = []
                for j in range(GROUP_ROWS):
                    gathers[j].wait()
                    scatters.append(
                        pltpu.async_copy(rows.at[j], acc.at[dbuf.at[j]], ssem,
                                         add=True))
                for h in scatters:
                    h.wait()
                return 0

            lax.fori_loop(0, n8, group, 0)

        plsc.subcore_barrier()

        def wb(k, _):
            off = stripe0 + k * LANES
            pltpu.sync_copy(acc.at[pl.ds(off, LANES)],
                            out_hbm.at[c].at[pl.ds(p * M_SPLIT + off, LANES)])
            return 0

        lax.fori_loop(0, STRIPE_CHUNKS, wb, 0)
        plsc.subcore_barrier()


_agg = pl.kernel(
    _agg_body,
    out_type=jax.ShapeDtypeStruct((2, M_SPLIT + ACC_ROWS, HALF), jnp.float32),
    mesh=_MESH,
    scratch_types=[
        pltpu.VMEM((GROUP_ROWS, LANES), jnp.int32),
        pltpu.VMEM((GROUP_ROWS, LANES), jnp.int32),
        pltpu.VMEM((GROUP_ROWS, LANES, HALF), jnp.float32),
        pltpu.VMEM((LANES, HALF), jnp.float32),
        pltpu.VMEM_SHARED((N_WORKERS, 16), jnp.int32),
        pltpu.SMEM((N_WORKERS, 16), jnp.int32),
        pltpu.VMEM_SHARED((ACC_ROWS, HALF), jnp.float32),
        pltpu.SemaphoreType.DMA,
        pltpu.SemaphoreType.DMA,
    ],
    compiler_params=pltpu.CompilerParams(use_tc_tiling_on_sc=False),
)


def _deg_body(dst_hbm, out_hbm, dbuf, obuf, acc, ssem):
    c = lax.axis_index("c")
    t = lax.axis_index("s")

    one = jnp.ones((16,), jnp.float32)
    z = jnp.zeros((16,), jnp.float32)
    stripe0 = t * DEG_STRIPE

    def zb(i, _):
        obuf[pl.ds(i * 16, 16)] = z
        return 0

    lax.fori_loop(0, LANES // 16, zb, 0)

    def zacc(k, _):
        pltpu.sync_copy(obuf, acc.at[pl.ds(stripe0 + k * LANES, LANES)])
        return 0

    lax.fori_loop(0, DEG_STRIPE_CHUNKS, zacc, 0)

    def ofill(i, _):
        obuf[pl.ds(i * 16, 16)] = one
        return 0

    lax.fori_loop(0, LANES // 16, ofill, 0)
    plsc.subcore_barrier()

    row0 = c * DEG_ROWS_PER_CORE + t * DEG_ROWS_PER_TILE

    def group(g, _):
        r = row0 + g * GROUP_ROWS
        pltpu.sync_copy(dst_hbm.at[pl.ds(r, GROUP_ROWS)], dbuf)
        scatters = [
            pltpu.async_copy(obuf, acc.at[dbuf.at[j]], ssem, add=True)
            for j in range(GROUP_ROWS)
        ]
        for h in scatters:
            h.wait()
        return 0

    lax.fori_loop(0, DEG_GROUPS, group, 0)

    plsc.subcore_barrier()

    def wb(k, _):
        off = stripe0 + k * LANES
        pltpu.sync_copy(acc.at[pl.ds(off, LANES)], out_hbm.at[c].at[pl.ds(off, LANES)])
        return 0

    lax.fori_loop(0, DEG_STRIPE_CHUNKS, wb, 0)


_deg = pl.kernel(
    _deg_body,
    out_type=jax.ShapeDtypeStruct((2, DEG_ROWS), jnp.float32),
    mesh=_MESH,
    scratch_types=[
        pltpu.VMEM((GROUP_ROWS, LANES), jnp.int32),
        pltpu.VMEM((LANES,), jnp.float32),
        pltpu.VMEM_SHARED((DEG_ROWS,), jnp.float32),
        pltpu.SemaphoreType.DMA,
    ],
)


_BLK = 4096
_GRID = (N_NODES + _BLK - 1) // _BLK


def _pre_body(deg0_ref, deg1_ref, x_ref, w1_ref, dis_ref, h1_ref, gt_ref):
    deg = deg0_ref[...] + deg1_ref[...] + 1.0
    dis = lax.rsqrt(deg)
    h = jnp.dot(x_ref[...], w1_ref[...], preferred_element_type=jnp.float32)
    g = h * dis
    dis_ref[...] = dis
    h1_ref[...] = h
    gt_ref[0] = g[:, :HALF]
    gt_ref[1] = g[:, HALF:]


_pre = pl.pallas_call(
    _pre_body,
    grid=(_GRID,),
    in_specs=[
        pl.BlockSpec((_BLK, 1), lambda i: (i, 0)),
        pl.BlockSpec((_BLK, 1), lambda i: (i, 0)),
        pl.BlockSpec((_BLK, IN_DIM), lambda i: (i, 0)),
        pl.BlockSpec((IN_DIM, HID), lambda i: (0, 0)),
    ],
    out_specs=[
        pl.BlockSpec((_BLK, 1), lambda i: (i, 0)),
        pl.BlockSpec((_BLK, HID), lambda i: (i, 0)),
        pl.BlockSpec((2, _BLK, HALF), lambda i: (0, i, 0)),
    ],
    out_shape=[
        jax.ShapeDtypeStruct((N_NODES, 1), jnp.float32),
        jax.ShapeDtypeStruct((N_NODES, HID), jnp.float32),
        jax.ShapeDtypeStruct((2, N_NODES, HALF), jnp.float32),
    ],
)


def _mid_body(acc_ref, h1_ref, dis_ref, w2_ref, b1_ref, h2_ref, gt2_ref):
    dis = dis_ref[...]
    agg = jnp.concatenate([acc_ref[0], acc_ref[1]], axis=1)
    out1 = jnp.maximum(agg * dis + h1_ref[...] * (dis * dis) + b1_ref[...], 0.0)
    h2 = jnp.dot(out1, w2_ref[...], preferred_element_type=jnp.float32)
    g2 = h2 * dis
    h2_ref[...] = h2
    gt2_ref[0] = g2[:, :HALF]
    gt2_ref[1] = g2[:, HALF:]


_mid = pl.pallas_call(
    _mid_body,
    grid=(_GRID,),
    in_specs=[
        pl.BlockSpec((2, _BLK, HALF), lambda i: (0, i, 0)),
        pl.BlockSpec((_BLK, HID), lambda i: (i, 0)),
        pl.BlockSpec((_BLK, 1), lambda i: (i, 0)),
        pl.BlockSpec((HID, HID), lambda i: (0, 0)),
        pl.BlockSpec((1, HID), lambda i: (0, 0)),
    ],
    out_specs=[
        pl.BlockSpec((_BLK, HID), lambda i: (i, 0)),
        pl.BlockSpec((2, _BLK, HALF), lambda i: (0, i, 0)),
    ],
    out_shape=[
        jax.ShapeDtypeStruct((N_NODES, HID), jnp.float32),
        jax.ShapeDtypeStruct((2, N_NODES, HALF), jnp.float32),
    ],
)


def _post_body(acc_ref, h2_ref, dis_ref, b2_ref, wp_ref, bp_ref, mask_ref,
               logit_ref):
    dis = dis_ref[...]
    agg = jnp.concatenate([acc_ref[0], acc_ref[1]], axis=1)
    out2 = jnp.maximum(agg * dis + h2_ref[...] * (dis * dis) + b2_ref[...], 0.0)
    z = jnp.sum(out2 * wp_ref[...], axis=1, keepdims=True) + bp_ref[0, 0]
    logit_ref[...] = jnp.where(mask_ref[...] > 0, z, jnp.float32(-1e9))


_post = pl.pallas_call(
    _post_body,
    grid=(_GRID,),
    in_specs=[
        pl.BlockSpec((2, _BLK, HALF), lambda i: (0, i, 0)),
        pl.BlockSpec((_BLK, HID), lambda i: (i, 0)),
        pl.BlockSpec((_BLK, 1), lambda i: (i, 0)),
        pl.BlockSpec((1, HID), lambda i: (0, 0)),
        pl.BlockSpec((1, HID), lambda i: (0, 0)),
        pl.BlockSpec((1, 1), lambda i: (0, 0)),
        pl.BlockSpec((_BLK, 1), lambda i: (i, 0)),
    ],
    out_specs=[pl.BlockSpec((_BLK, 1), lambda i: (i, 0))],
    out_shape=[jax.ShapeDtypeStruct((N_NODES, 1), jnp.float32)],
)


def _softmax_body(x_ref, o_ref):
    x = x_ref[...]
    m = jnp.max(x)
    e = jnp.exp(x - m)
    o_ref[...] = e / jnp.sum(e)


_softmax = pl.pallas_call(
    _softmax_body,
    out_shape=jax.ShapeDtypeStruct((1, N_NODES), jnp.float32),
)


def kernel(x, edge_index, ready_mask, W1, b1, W2, b2, Wp, bp):
    src = edge_index[0]
    dst = edge_index[1]
    pad = E_PAD - E_EDGES
    srcp = jnp.concatenate([src, jnp.zeros((pad,), src.dtype)]).reshape(E_ROWS, LANES)
    dstp = jnp.concatenate([dst, jnp.full((pad,), N_NODES, dst.dtype)]).reshape(E_ROWS, LANES)

    degout = _deg(dstp)
    deg0 = degout[0, :N_NODES].reshape(N_NODES, 1)
    deg1 = degout[1, :N_NODES].reshape(N_NODES, 1)

    osrc, odst, cnts = _part(srcp, dstp)

    dis, h1, gt1 = _pre(deg0, deg1, x, W1)
    acc1 = _agg(osrc, odst, cnts, gt1)
    h2, gt2 = _mid(acc1, h1, dis, W2, b1.reshape(1, HID))
    acc2 = _agg(osrc, odst, cnts, gt2)

    mask = ready_mask.astype(jnp.int32).reshape(N_NODES, 1)
    (logits,) = _post(acc2, h2, dis, b2.reshape(1, HID),
                      Wp.reshape(1, HID), bp.reshape(1, 1), mask)
    probs = _softmax(logits.reshape(1, N_NODES))
    return probs.reshape(N_NODES)


# speedup vs baseline: 4.7781x; 1.1208x over previous
import jax
import jax.numpy as jnp
from jax import lax
from jax.experimental import pallas as pl
from jax.experimental.pallas import tpu as pltpu
from jax.experimental.pallas import tpu_sc as plsc

N_NODES = 100000
E_EDGES = 1600000
IN_DIM = 6
HID = 32
HALF = 16

LANES = 128
E_PAD = 1638400
E_ROWS = E_PAD // LANES
N_TILES = 16

M_SPLIT = 50176
ACC_ROWS = 51200
STRIPE = ACC_ROWS // N_TILES
STRIPE_CHUNKS = STRIPE // LANES

GROUP_ROWS = 8
ROWS_PER_TILE = E_ROWS // N_TILES
N_GROUPS = ROWS_PER_TILE // GROUP_ROWS

DEG_ROWS = 100352
DEG_STRIPE = DEG_ROWS // N_TILES
DEG_STRIPE_CHUNKS = DEG_STRIPE // LANES
DEG_ROWS_PER_CORE = E_ROWS // 2
DEG_ROWS_PER_TILE = DEG_ROWS_PER_CORE // N_TILES
DEG_GROUPS = DEG_ROWS_PER_TILE // GROUP_ROWS

_MESH = plsc.VectorSubcoreMesh(core_axis_name="c", subcore_axis_name="s")


N_WORKERS = 32
PART_ROWS_PER_WORKER = E_ROWS // N_WORKERS
PART_GROUPS = PART_ROWS_PER_WORKER // GROUP_ROWS
SLOT_ROWS = PART_ROWS_PER_WORKER + 8
STAGE_CAP = 1280
GARBAGE = M_SPLIT


def _part_body(src_hbm, dst_hbm, osrc_hbm, odst_hbm, cnt_hbm,
               sbuf, dbuf, st_s0, st_d0, st_s1, st_d1, cbuf):
    c = lax.axis_index("c")
    t = lax.axis_index("s")
    w = c * 16 + t
    row0 = w * PART_ROWS_PER_WORKER
    stages = ((st_s0, st_d0), (st_s1, st_d1))

    def flush(p, st_s, st_d, cnt, rows):
        nf = cnt >> 7

        def fl(k, _):
            pltpu.sync_copy(st_s.at[pl.ds(k * LANES, LANES)],
                            osrc_hbm.at[p, w, rows + k])
            pltpu.sync_copy(st_d.at[pl.ds(k * LANES, LANES)],
                            odst_hbm.at[p, w, rows + k])
            return 0

        lax.fori_loop(0, nf, fl, 0)
        off = nf * LANES
        for i in range(8):
            sv = st_s[pl.ds(off + i * 16, 16)]
            dv = st_d[pl.ds(off + i * 16, 16)]
            st_s[pl.ds(i * 16, 16)] = sv
            st_d[pl.ds(i * 16, 16)] = dv
        return cnt - (nf << 7), rows + nf

    def group(g, carry):
        cnt0, rows0, cnt1, rows1 = carry
        r = row0 + g * GROUP_ROWS
        pltpu.sync_copy(src_hbm.at[pl.ds(r, GROUP_ROWS)], sbuf)
        pltpu.sync_copy(dst_hbm.at[pl.ds(r, GROUP_ROWS)], dbuf)
        for j in range(GROUP_ROWS):
            def lanes16(k, cc):
                c0, c1 = cc
                sl = pl.ds(k * 16, 16)
                s16 = sbuf[j, sl]
                d16 = dbuf[j, sl]
                m0 = d16 < M_SPLIT
                m1 = jnp.logical_not(m0)
                d16b = d16 - M_SPLIT
                plsc.store_compressed(st_s0.at[pl.ds(c0, 16)], s16, mask=m0)
                plsc.store_compressed(st_d0.at[pl.ds(c0, 16)], d16, mask=m0)
                plsc.store_compressed(st_s1.at[pl.ds(c1, 16)], s16, mask=m1)
                plsc.store_compressed(st_d1.at[pl.ds(c1, 16)], d16b, mask=m1)
                n0 = plsc.all_reduce_population_count(m0)[0]
                return c0 + n0, c1 + (16 - n0)

            cnt0, cnt1 = lax.fori_loop(0, LANES // 16, lanes16, (cnt0, cnt1))
        cnt0, rows0 = flush(0, st_s0, st_d0, cnt0, rows0)
        cnt1, rows1 = flush(1, st_s1, st_d1, cnt1, rows1)
        return cnt0, rows0, cnt1, rows1

    cnt0, rows0, cnt1, rows1 = lax.fori_loop(
        0, PART_GROUPS, group, (jnp.int32(0), jnp.int32(0), jnp.int32(0), jnp.int32(0)))

    zero16 = jnp.zeros((16,), jnp.int32)
    garb16 = jnp.full((16,), GARBAGE, jnp.int32)
    final = []
    for p, (st_s, st_d) in enumerate(stages):
        cnt, rows = (cnt0, rows0) if p == 0 else (cnt1, rows1)
        for i in range(8):
            st_s[pl.ds(cnt + i * 16, 16)] = zero16
            st_d[pl.ds(cnt + i * 16, 16)] = garb16
        pltpu.sync_copy(st_s.at[pl.ds(0, LANES)], osrc_hbm.at[p, w, rows])
        pltpu.sync_copy(st_d.at[pl.ds(0, LANES)], odst_hbm.at[p, w, rows])
        rows = rows + 1
        for i in range(8):
            st_s[pl.ds(i * 16, 16)] = zero16
            st_d[pl.ds(i * 16, 16)] = garb16
        nextra = (-rows) & 7

        def pad(k, _):
            pltpu.sync_copy(st_s.at[pl.ds(0, LANES)], osrc_hbm.at[p, w, rows + k])
            pltpu.sync_copy(st_d.at[pl.ds(0, LANES)], odst_hbm.at[p, w, rows + k])
            return 0

        lax.fori_loop(0, nextra, pad, 0)
        final.append((rows + nextra) >> 3)

    idx16 = lax.iota(jnp.int32, 16)
    cvec = jnp.where(idx16 == 0, final[0], jnp.where(idx16 == 1, final[1], 0))
    cbuf[...] = cvec
    pltpu.sync_copy(cbuf, cnt_hbm.at[w])


_part = pl.kernel(
    _part_body,
    out_type=(
        jax.ShapeDtypeStruct((2, N_WORKERS, SLOT_ROWS, LANES), jnp.int32),
        jax.ShapeDtypeStruct((2, N_WORKERS, SLOT_ROWS, LANES), jnp.int32),
        jax.ShapeDtypeStruct((N_WORKERS, 16), jnp.int32),
    ),
    mesh=_MESH,
    scratch_types=[
        pltpu.VMEM((GROUP_ROWS, LANES), jnp.int32),
        pltpu.VMEM((GROUP_ROWS, LANES), jnp.int32),
        pltpu.VMEM((STAGE_CAP,), jnp.int32),
        pltpu.VMEM((STAGE_CAP,), jnp.int32),
        pltpu.VMEM((STAGE_CAP,), jnp.int32),
        pltpu.VMEM((STAGE_CAP,), jnp.int32),
        pltpu.VMEM((16,), jnp.int32),
    ],
    compiler_params=pltpu.CompilerParams(use_tc_tiling_on_sc=False,
                                         needs_layout_passes=False),
)


def _agg_body(osrc_hbm, odst_hbm, cnt_hbm, gt_hbm, out_hbm,
              sbuf, dbuf, rows, zbuf, cnt_spm, cnt_smem, acc, esem, gsem, ssem):
    c = lax.axis_index("c")
    t = lax.axis_index("s")

    pltpu.sync_copy(cnt_hbm, cnt_spm)
    pltpu.sync_copy(cnt_spm, cnt_smem)

    z = jnp.zeros((16,), jnp.float32)

    def zfill(i, _):
        zbuf[i, :] = z
        return 0

    lax.fori_loop(0, LANES, zfill, 0)

    gt_c = gt_hbm.at[c]
    stripe0 = t * STRIPE

    for p in (0, 1):
        def zacc(k, _):
            pltpu.sync_copy(zbuf, acc.at[pl.ds(stripe0 + k * LANES, LANES)])
            return 0

        lax.fori_loop(0, STRIPE_CHUNKS, zacc, 0)
        plsc.subcore_barrier()

        for si in range(2):
            w = t * 2 + si
            n8 = cnt_smem[w, p]

            def eload(g, bank):
                r = g * GROUP_ROWS
                pltpu.async_copy(osrc_hbm.at[p, w, pl.ds(r, GROUP_ROWS)],
                                 sbuf.at[bank], esem)
                pltpu.async_copy(odst_hbm.at[p, w, pl.ds(r, GROUP_ROWS)],
                                 dbuf.at[bank], esem)

            def ewait(bank):
                pltpu.make_async_copy(osrc_hbm.at[p, w, pl.ds(0, GROUP_ROWS)],
                                      sbuf.at[bank], esem).wait()
                pltpu.make_async_copy(odst_hbm.at[p, w, pl.ds(0, GROUP_ROWS)],
                                      dbuf.at[bank], esem).wait()

            @pl.when(n8 > 0)
            def _():
                eload(0, 0)

            def group(g, _):
                be = lax.rem(g, 3)
                br = jnp.bitwise_and(g, 1)
                ewait(be)

                @pl.when(g >= 2)
                def _():
                    for j in range(GROUP_ROWS):
                        pltpu.make_async_copy(
                            rows.at[br, j], acc.at[dbuf.at[be, j]],
                            ssem).wait()

                @pl.when(g + 1 < n8)
                def _():
                    eload(g + 1, lax.rem(g + 1, 3))

                gathers = [
                    pltpu.async_copy(gt_c.at[sbuf.at[be, j]],
                                     rows.at[br, j], gsem)
                    for j in range(GROUP_ROWS)
                ]
                for j in range(GROUP_ROWS):
                    gathers[j].wait()
                    pltpu.async_copy(rows.at[br, j], acc.at[dbuf.at[be, j]],
                                     ssem, add=True)
                return 0

            lax.fori_loop(0, n8, group, 0)

            def drain(k, _):
                pltpu.make_async_copy(rows.at[0, 0], acc.at[dbuf.at[0, 0]],
                                      ssem).wait()
                return 0

            lax.fori_loop(0, jnp.minimum(n8, 2) * GROUP_ROWS, drain, 0)

        plsc.subcore_barrier()

        def wb(k, _):
            off = stripe0 + k * LANES
            pltpu.sync_copy(acc.at[pl.ds(off, LANES)],
                            out_hbm.at[c].at[pl.ds(p * M_SPLIT + off, LANES)])
            return 0

        lax.fori_loop(0, STRIPE_CHUNKS, wb, 0)
        plsc.subcore_barrier()


_agg = pl.kernel(
    _agg_body,
    out_type=jax.ShapeDtypeStruct((2, M_SPLIT + ACC_ROWS, HALF), jnp.float32),
    mesh=_MESH,
    scratch_types=[
        pltpu.VMEM((3, GROUP_ROWS, LANES), jnp.int32),
        pltpu.VMEM((3, GROUP_ROWS, LANES), jnp.int32),
        pltpu.VMEM((2, GROUP_ROWS, LANES, HALF), jnp.float32),
        pltpu.VMEM((LANES, HALF), jnp.float32),
        pltpu.VMEM_SHARED((N_WORKERS, 16), jnp.int32),
        pltpu.SMEM((N_WORKERS, 16), jnp.int32),
        pltpu.VMEM_SHARED((ACC_ROWS, HALF), jnp.float32),
        pltpu.SemaphoreType.DMA,
        pltpu.SemaphoreType.DMA,
        pltpu.SemaphoreType.DMA,
    ],
    compiler_params=pltpu.CompilerParams(use_tc_tiling_on_sc=False),
)


def _deg_body(dst_hbm, out_hbm, dbuf, obuf, acc, ssem):
    c = lax.axis_index("c")
    t = lax.axis_index("s")

    one = jnp.ones((16,), jnp.float32)
    z = jnp.zeros((16,), jnp.float32)
    stripe0 = t * DEG_STRIPE

    def zb(i, _):
        obuf[pl.ds(i * 16, 16)] = z
        return 0

    lax.fori_loop(0, LANES // 16, zb, 0)

    def zacc(k, _):
        pltpu.sync_copy(obuf, acc.at[pl.ds(stripe0 + k * LANES, LANES)])
        return 0

    lax.fori_loop(0, DEG_STRIPE_CHUNKS, zacc, 0)

    def ofill(i, _):
        obuf[pl.ds(i * 16, 16)] = one
        return 0

    lax.fori_loop(0, LANES // 16, ofill, 0)
    plsc.subcore_barrier()

    row0 = c * DEG_ROWS_PER_CORE + t * DEG_ROWS_PER_TILE

    def group(g, _):
        r = row0 + g * GROUP_ROWS
        pltpu.sync_copy(dst_hbm.at[pl.ds(r, GROUP_ROWS)], dbuf)
        scatters = [
            pltpu.async_copy(obuf, acc.at[dbuf.at[j]], ssem, add=True)
            for j in range(GROUP_ROWS)
        ]
        for h in scatters:
            h.wait()
        return 0

    lax.fori_loop(0, DEG_GROUPS, group, 0)

    plsc.subcore_barrier()

    def wb(k, _):
        off = stripe0 + k * LANES
        pltpu.sync_copy(acc.at[pl.ds(off, LANES)], out_hbm.at[c].at[pl.ds(off, LANES)])
        return 0

    lax.fori_loop(0, DEG_STRIPE_CHUNKS, wb, 0)


_deg = pl.kernel(
    _deg_body,
    out_type=jax.ShapeDtypeStruct((2, DEG_ROWS), jnp.float32),
    mesh=_MESH,
    scratch_types=[
        pltpu.VMEM((GROUP_ROWS, LANES), jnp.int32),
        pltpu.VMEM((LANES,), jnp.float32),
        pltpu.VMEM_SHARED((DEG_ROWS,), jnp.float32),
        pltpu.SemaphoreType.DMA,
    ],
)


_BLK = 4096
_GRID = (N_NODES + _BLK - 1) // _BLK


def _pre_body(deg0_ref, deg1_ref, x_ref, w1_ref, dis_ref, h1_ref, gt_ref):
    deg = deg0_ref[...] + deg1_ref[...] + 1.0
    dis = lax.rsqrt(deg)
    h = jnp.dot(x_ref[...], w1_ref[...], preferred_element_type=jnp.float32)
    g = h * dis
    dis_ref[...] = dis
    h1_ref[...] = h
    gt_ref[0] = g[:, :HALF]
    gt_ref[1] = g[:, HALF:]


_pre = pl.pallas_call(
    _pre_body,
    grid=(_GRID,),
    in_specs=[
        pl.BlockSpec((_BLK, 1), lambda i: (i, 0)),
        pl.BlockSpec((_BLK, 1), lambda i: (i, 0)),
        pl.BlockSpec((_BLK, IN_DIM), lambda i: (i, 0)),
        pl.BlockSpec((IN_DIM, HID), lambda i: (0, 0)),
    ],
    out_specs=[
        pl.BlockSpec((_BLK, 1), lambda i: (i, 0)),
        pl.BlockSpec((_BLK, HID), lambda i: (i, 0)),
        pl.BlockSpec((2, _BLK, HALF), lambda i: (0, i, 0)),
    ],
    out_shape=[
        jax.ShapeDtypeStruct((N_NODES, 1), jnp.float32),
        jax.ShapeDtypeStruct((N_NODES, HID), jnp.float32),
        jax.ShapeDtypeStruct((2, N_NODES, HALF), jnp.float32),
    ],
)


def _mid_body(acc_ref, h1_ref, dis_ref, w2_ref, b1_ref, h2_ref, gt2_ref):
    dis = dis_ref[...]
    agg = jnp.concatenate([acc_ref[0], acc_ref[1]], axis=1)
    out1 = jnp.maximum(agg * dis + h1_ref[...] * (dis * dis) + b1_ref[...], 0.0)
    h2 = jnp.dot(out1, w2_ref[...], preferred_element_type=jnp.float32)
    g2 = h2 * dis
    h2_ref[...] = h2
    gt2_ref[0] = g2[:, :HALF]
    gt2_ref[1] = g2[:, HALF:]


_mid = pl.pallas_call(
    _mid_body,
    grid=(_GRID,),
    in_specs=[
        pl.BlockSpec((2, _BLK, HALF), lambda i: (0, i, 0)),
        pl.BlockSpec((_BLK, HID), lambda i: (i, 0)),
        pl.BlockSpec((_BLK, 1), lambda i: (i, 0)),
        pl.BlockSpec((HID, HID), lambda i: (0, 0)),
        pl.BlockSpec((1, HID), lambda i: (0, 0)),
    ],
    out_specs=[
        pl.BlockSpec((_BLK, HID), lambda i: (i, 0)),
        pl.BlockSpec((2, _BLK, HALF), lambda i: (0, i, 0)),
    ],
    out_shape=[
        jax.ShapeDtypeStruct((N_NODES, HID), jnp.float32),
        jax.ShapeDtypeStruct((2, N_NODES, HALF), jnp.float32),
    ],
)


def _post_body(acc_ref, h2_ref, dis_ref, b2_ref, wp_ref, bp_ref, mask_ref,
               logit_ref):
    dis = dis_ref[...]
    agg = jnp.concatenate([acc_ref[0], acc_ref[1]], axis=1)
    out2 = jnp.maximum(agg * dis + h2_ref[...] * (dis * dis) + b2_ref[...], 0.0)
    z = jnp.sum(out2 * wp_ref[...], axis=1, keepdims=True) + bp_ref[0, 0]
    logit_ref[...] = jnp.where(mask_ref[...] > 0, z, jnp.float32(-1e9))


_post = pl.pallas_call(
    _post_body,
    grid=(_GRID,),
    in_specs=[
        pl.BlockSpec((2, _BLK, HALF), lambda i: (0, i, 0)),
        pl.BlockSpec((_BLK, HID), lambda i: (i, 0)),
        pl.BlockSpec((_BLK, 1), lambda i: (i, 0)),
        pl.BlockSpec((1, HID), lambda i: (0, 0)),
        pl.BlockSpec((1, HID), lambda i: (0, 0)),
        pl.BlockSpec((1, 1), lambda i: (0, 0)),
        pl.BlockSpec((_BLK, 1), lambda i: (i, 0)),
    ],
    out_specs=[pl.BlockSpec((_BLK, 1), lambda i: (i, 0))],
    out_shape=[jax.ShapeDtypeStruct((N_NODES, 1), jnp.float32)],
)


def _softmax_body(x_ref, o_ref):
    x = x_ref[...]
    m = jnp.max(x)
    e = jnp.exp(x - m)
    o_ref[...] = e / jnp.sum(e)


_softmax = pl.pallas_call(
    _softmax_body,
    out_shape=jax.ShapeDtypeStruct((1, N_NODES), jnp.float32),
)


def kernel(x, edge_index, ready_mask, W1, b1, W2, b2, Wp, bp):
    src = edge_index[0]
    dst = edge_index[1]
    pad = E_PAD - E_EDGES
    srcp = jnp.concatenate([src, jnp.zeros((pad,), src.dtype)]).reshape(E_ROWS, LANES)
    dstp = jnp.concatenate([dst, jnp.full((pad,), N_NODES, dst.dtype)]).reshape(E_ROWS, LANES)

    degout = _deg(dstp)
    deg0 = degout[0, :N_NODES].reshape(N_NODES, 1)
    deg1 = degout[1, :N_NODES].reshape(N_NODES, 1)

    osrc, odst, cnts = _part(srcp, dstp)

    dis, h1, gt1 = _pre(deg0, deg1, x, W1)
    acc1 = _agg(osrc, odst, cnts, gt1)
    h2, gt2 = _mid(acc1, h1, dis, W2, b1.reshape(1, HID))
    acc2 = _agg(osrc, odst, cnts, gt2)

    mask = ready_mask.astype(jnp.int32).reshape(N_NODES, 1)
    (logits,) = _post(acc2, h2, dis, b2.reshape(1, HID),
                      Wp.reshape(1, HID), bp.reshape(1, 1), mask)
    probs = _softmax(logits.reshape(1, N_NODES))
    return probs.reshape(N_NODES)
